# Initial kernel scaffold; baseline (speedup 1.0000x reference)
import jax
import jax.numpy as jnp
from jax.experimental import pallas as pl


def kernel(voxel_coords, voxel_features, W_proj, b_proj, W_pe1, b_pe1, W_pe2, b_pe2, Wq, bq, Wk, bk, Wv, bv, Wo, bo, W_f1, b_f1, W_f2, b_f2, ln_g, ln_b, W_fu1, b_fu1, W_fu2, b_fu2, bn_g, bn_b):
    raise NotImplementedError("write your pallas kernel here")


# speedup vs baseline: 2.2302x; 2.2302x over previous
import functools

import jax
import jax.numpy as jnp
from jax import lax
from jax.experimental import pallas as pl
from jax.experimental.pallas import tpu as pltpu
from jax.experimental.pallas import tpu_sc as plsc

N = 8192
D_CHL = 16
D_MODEL = 64
N_HEADS = 4
HEAD_DIM = D_MODEL // N_HEADS
D_FFN = 128
M = 16
DIST = 2.0
INF = 1e9

BQ = 128
BN = 256
CH = 128


def _src_body(feat_ref, crt_ref, wp_ref, bp_ref, w1_ref, b1_ref, w2_ref,
              b2_ref, src_ref):
    nc = crt_ref[...] * (1.0 / 399.0)
    h = jnp.maximum(
        jnp.dot(nc, w1_ref[...], preferred_element_type=jnp.float32)
        + b1_ref[...], 0.0)
    pe = jnp.dot(h, w2_ref[...], preferred_element_type=jnp.float32) + b2_ref[...]
    src_ref[...] = (
        jnp.dot(feat_ref[...], wp_ref[...], preferred_element_type=jnp.float32)
        + bp_ref[...] + pe)


def _knn_body(cq_ref, ct_ref, idx_ref, val_ref):
    cq = cq_ref[...]
    ct = ct_ref[...]
    man = jnp.abs(cq[:, 0:1] - ct[0:1, :])
    man = man + jnp.abs(cq[:, 1:2] - ct[1:2, :])
    man = man + jnp.abs(cq[:, 2:3] - ct[2:3, :])
    j_iota = lax.broadcasted_iota(jnp.int32, (BQ, N), 1).astype(jnp.float32)
    keys = jnp.where(man <= DIST, man * float(N) + j_iota, INF)
    for t in range(M):
        m = jnp.min(keys, axis=1, keepdims=True)
        validt = m < (0.5 * INF)
        d = jnp.floor(m * (1.0 / float(N)))
        jf = m - d * float(N)
        idx_ref[:, t:t + 1] = jnp.where(validt, jf, 0.0).astype(jnp.int32)
        val_ref[:, t:t + 1] = validt.astype(jnp.float32)
        keys = jnp.where(keys == m, INF, keys)


def _gather_rows(table, idx_flat):
    info = plsc.get_sparse_core_info()
    nw = info.num_cores * info.num_subcores
    b = idx_flat.shape[0]
    b_per_w = b // nw
    n_ch = b_per_w // CH
    mesh = plsc.VectorSubcoreMesh(core_axis_name="c", subcore_axis_name="s")

    @functools.partial(
        pl.kernel, mesh=mesh,
        compiler_params=pltpu.CompilerParams(use_tc_tiling_on_sc=False),
        out_type=jax.ShapeDtypeStruct((b, D_MODEL), jnp.float32),
        scratch_types=[
            pltpu.VMEM((CH,), jnp.int32),
            pltpu.VMEM((CH, D_MODEL), jnp.float32),
            pltpu.SemaphoreType.DMA,
        ],
    )
    def gk(table_hbm, idx_hbm, out_hbm, idx_v, rows_v, sem):
        wid = lax.axis_index("s") * info.num_cores + lax.axis_index("c")
        base = wid * b_per_w

        def body(c, carry):
            off = base + c * CH
            pltpu.sync_copy(idx_hbm.at[pl.ds(off, CH)], idx_v)
            pltpu.async_copy(table_hbm.at[idx_v], rows_v, sem).wait()
            pltpu.sync_copy(rows_v, out_hbm.at[pl.ds(off, CH)])
            return carry

        lax.fori_loop(0, n_ch, body, 0)

    return gk(table, idx_flat)


def _attn_body(kv_ref, q_ref, vm_ref, feat_ref, wq_ref, bq_ref, wk_ref,
               bk_ref, wv_ref, bv_ref, wo_ref, bo_ref, wf1_ref, bf1_ref,
               wf2_ref, bf2_ref, lng_ref, lnb_ref, wfu1_ref, bfu1_ref,
               wfu2a_ref, wfu2b_ref, bfu2_ref, out_ref):
    kvm = kv_ref[...] * vm_ref[...][:, :, None]
    q_rows = q_ref[...]
    q = jnp.dot(q_rows, wq_ref[...], preferred_element_type=jnp.float32) + bq_ref[...]
    kv2 = kvm.reshape(M * BN, D_MODEL)
    k = jnp.dot(kv2, wk_ref[...], preferred_element_type=jnp.float32) + bk_ref[...]
    v = jnp.dot(kv2, wv_ref[...], preferred_element_type=jnp.float32) + bv_ref[...]
    k4 = k.reshape(M, BN, N_HEADS, HEAD_DIM)
    v4 = v.reshape(M, BN, N_HEADS, HEAD_DIM)
    q4 = q.reshape(1, BN, N_HEADS, HEAD_DIM)
    scores = (q4 * k4).sum(-1) * (1.0 / 4.0)
    smax = scores.max(axis=0, keepdims=True)
    e = jnp.exp(scores - smax)
    w = e / e.sum(axis=0, keepdims=True)
    attn = (w[:, :, :, None] * v4).sum(axis=0)
    o = jnp.dot(attn.reshape(BN, D_MODEL), wo_ref[...],
                preferred_element_type=jnp.float32) + bo_ref[...]
    h1 = jnp.maximum(
        jnp.dot(o, wf1_ref[...], preferred_element_type=jnp.float32)
        + bf1_ref[...], 0.0)
    t2 = jnp.dot(h1, wf2_ref[...], preferred_element_type=jnp.float32) + bf2_ref[...]
    t = o + t2
    mu = t.mean(axis=1, keepdims=True)
    var = ((t - mu) ** 2).mean(axis=1, keepdims=True)
    tgt = (t - mu) / jnp.sqrt(var + 1e-5) * lng_ref[...] + lnb_ref[...]
    tf = jnp.dot(tgt, wfu1_ref[...], preferred_element_type=jnp.float32) + bfu1_ref[...]
    out_ref[...] = (
        jnp.dot(feat_ref[...], wfu2a_ref[...], preferred_element_type=jnp.float32)
        + jnp.dot(tf, wfu2b_ref[...], preferred_element_type=jnp.float32)
        + bfu2_ref[...])


def _bn_body(x_ref, g_ref, b_ref, out_ref):
    x = x_ref[...]
    mu = x.mean(axis=0, keepdims=True)
    var = ((x - mu) ** 2).mean(axis=0, keepdims=True)
    y = (x - mu) / jnp.sqrt(var + 1e-5) * g_ref[...] + b_ref[...]
    out_ref[...] = jnp.maximum(y, 0.0)


def kernel(voxel_coords, voxel_features, W_proj, b_proj, W_pe1, b_pe1, W_pe2,
           b_pe2, Wq, bq, Wk, bk, Wv, bv, Wo, bo, W_f1, b_f1, W_f2, b_f2,
           ln_g, ln_b, W_fu1, b_fu1, W_fu2, b_fu2, bn_g, bn_b):
    crt = voxel_coords.astype(jnp.float32)
    ct = crt.T
    r2 = lambda a: a.reshape(1, -1)

    src = pl.pallas_call(
        _src_body,
        out_shape=jax.ShapeDtypeStruct((N, D_MODEL), jnp.float32),
    )(voxel_features, crt, W_proj, r2(b_proj), W_pe1, r2(b_pe1), W_pe2,
      r2(b_pe2))

    idx, validf = pl.pallas_call(
        _knn_body,
        grid=(N // BQ,),
        in_specs=[
            pl.BlockSpec((BQ, 3), lambda i: (i, 0)),
            pl.BlockSpec((3, N), lambda i: (0, 0)),
        ],
        out_specs=[
            pl.BlockSpec((BQ, M), lambda i: (i, 0)),
            pl.BlockSpec((BQ, M), lambda i: (i, 0)),
        ],
        out_shape=[
            jax.ShapeDtypeStruct((N, M), jnp.int32),
            jax.ShapeDtypeStruct((N, M), jnp.float32),
        ],
    )(crt, ct)

    neigh_flat = _gather_rows(src, idx.reshape(-1))
    kv_view = neigh_flat.reshape(M, N, D_MODEL)
    q_view = neigh_flat.reshape(N, M * D_MODEL)[:, :D_MODEL]
    vm_kv = validf.reshape(-1).reshape(M, N)

    fused_pre = pl.pallas_call(
        _attn_body,
        grid=(N // BN,),
        in_specs=[
            pl.BlockSpec((M, BN, D_MODEL), lambda i: (0, i, 0)),
            pl.BlockSpec((BN, D_MODEL), lambda i: (i, 0)),
            pl.BlockSpec((M, BN), lambda i: (0, i)),
            pl.BlockSpec((BN, D_CHL), lambda i: (i, 0)),
            pl.BlockSpec((D_MODEL, D_MODEL), lambda i: (0, 0)),
            pl.BlockSpec((1, D_MODEL), lambda i: (0, 0)),
            pl.BlockSpec((D_MODEL, D_MODEL), lambda i: (0, 0)),
            pl.BlockSpec((1, D_MODEL), lambda i: (0, 0)),
            pl.BlockSpec((D_MODEL, D_MODEL), lambda i: (0, 0)),
            pl.BlockSpec((1, D_MODEL), lambda i: (0, 0)),
            pl.BlockSpec((D_MODEL, D_MODEL), lambda i: (0, 0)),
            pl.BlockSpec((1, D_MODEL), lambda i: (0, 0)),
            pl.BlockSpec((D_MODEL, D_FFN), lambda i: (0, 0)),
            pl.BlockSpec((1, D_FFN), lambda i: (0, 0)),
            pl.BlockSpec((D_FFN, D_MODEL), lambda i: (0, 0)),
            pl.BlockSpec((1, D_MODEL), lambda i: (0, 0)),
            pl.BlockSpec((1, D_MODEL), lambda i: (0, 0)),
            pl.BlockSpec((1, D_MODEL), lambda i: (0, 0)),
            pl.BlockSpec((D_MODEL, D_CHL), lambda i: (0, 0)),
            pl.BlockSpec((1, D_CHL), lambda i: (0, 0)),
            pl.BlockSpec((D_CHL, D_CHL), lambda i: (0, 0)),
            pl.BlockSpec((D_CHL, D_CHL), lambda i: (0, 0)),
            pl.BlockSpec((1, D_CHL), lambda i: (0, 0)),
        ],
        out_specs=pl.BlockSpec((BN, D_CHL), lambda i: (i, 0)),
        out_shape=jax.ShapeDtypeStruct((N, D_CHL), jnp.float32),
    )(kv_view, q_view, vm_kv, voxel_features, Wq, r2(bq), Wk, r2(bk), Wv,
      r2(bv), Wo, r2(bo), W_f1, r2(b_f1), W_f2, r2(b_f2), r2(ln_g), r2(ln_b),
      W_fu1, r2(b_fu1), W_fu2[:D_CHL], W_fu2[D_CHL:], r2(b_fu2))

    return pl.pallas_call(
        _bn_body,
        out_shape=jax.ShapeDtypeStruct((N, D_CHL), jnp.float32),
    )(fused_pre, r2(bn_g), r2(bn_b))


# speedup vs baseline: 2.2370x; 1.0031x over previous
import functools

import jax
import jax.numpy as jnp
from jax import lax
from jax.experimental import pallas as pl
from jax.experimental.pallas import tpu as pltpu
from jax.experimental.pallas import tpu_sc as plsc

N = 8192
D_CHL = 16
D_MODEL = 64
N_HEADS = 4
HEAD_DIM = D_MODEL // N_HEADS
D_FFN = 128
M = 16
DIST = 2.0
INF = 1e9

BQ = 128
BN = 256
CH = 128


def _src_body(feat_ref, crt_ref, wp_ref, bp_ref, w1_ref, b1_ref, w2_ref,
              b2_ref, src_ref):
    nc = crt_ref[...] * (1.0 / 399.0)
    h = jnp.maximum(
        jnp.dot(nc, w1_ref[...], preferred_element_type=jnp.float32)
        + b1_ref[...], 0.0)
    pe = jnp.dot(h, w2_ref[...], preferred_element_type=jnp.float32) + b2_ref[...]
    src_ref[...] = (
        jnp.dot(feat_ref[...], wp_ref[...], preferred_element_type=jnp.float32)
        + bp_ref[...] + pe)


def _knn_body(cq_ref, ct_ref, idx_ref, val_ref):
    cq = cq_ref[...]
    ct = ct_ref[...]
    man = jnp.abs(cq[:, 0:1] - ct[0:1, :])
    man = man + jnp.abs(cq[:, 1:2] - ct[1:2, :])
    man = man + jnp.abs(cq[:, 2:3] - ct[2:3, :])
    j_iota = lax.broadcasted_iota(jnp.int32, (BQ, N), 1).astype(jnp.float32)
    keys = jnp.where(man <= DIST, man * float(N) + j_iota, INF)
    for t in range(M):
        m = jnp.min(keys, axis=1, keepdims=True)
        validt = m < (0.5 * INF)
        d = jnp.floor(m * (1.0 / float(N)))
        jf = m - d * float(N)
        idx_ref[:, t:t + 1] = jnp.where(validt, jf, 0.0).astype(jnp.int32)
        val_ref[:, t:t + 1] = validt.astype(jnp.float32)
        keys = jnp.where(keys == m, INF, keys)


def _gather_rows(table, idx_flat):
    info = plsc.get_sparse_core_info()
    nw = info.num_cores * info.num_subcores
    b = idx_flat.shape[0]
    b_per_w = b // nw
    n_ch = b_per_w // CH
    mesh = plsc.VectorSubcoreMesh(core_axis_name="c", subcore_axis_name="s")

    grp = 1024 // CH
    n_grp = b_per_w // 1024

    @functools.partial(
        pl.kernel, mesh=mesh,
        compiler_params=pltpu.CompilerParams(use_tc_tiling_on_sc=False),
        out_type=jax.ShapeDtypeStruct((b, D_MODEL), jnp.float32),
        scratch_types=[
            pltpu.VMEM((b_per_w,), jnp.int32),
            pltpu.VMEM((1024, D_MODEL), jnp.float32),
            pltpu.SemaphoreType.DMA,
        ],
    )
    def gk(table_hbm, idx_hbm, out_hbm, idx_v, rows_v, sem):
        wid = lax.axis_index("s") * info.num_cores + lax.axis_index("c")
        base = wid * b_per_w
        pltpu.sync_copy(idx_hbm.at[pl.ds(base, b_per_w)], idx_v)

        def body(g, carry):
            copies = [
                pltpu.async_copy(
                    table_hbm.at[idx_v.at[pl.ds(g * 1024 + c * CH, CH)]],
                    rows_v.at[pl.ds(c * CH, CH)], sem)
                for c in range(grp)
            ]
            for cp in copies:
                cp.wait()
            pltpu.sync_copy(rows_v, out_hbm.at[pl.ds(base + g * 1024, 1024)])
            return carry

        lax.fori_loop(0, n_grp, body, 0)

    return gk(table, idx_flat)


def _attn_body(kv_ref, q_ref, vm_ref, feat_ref, wq_ref, bq_ref, wk_ref,
               bk_ref, wv_ref, bv_ref, wo_ref, bo_ref, wf1_ref, bf1_ref,
               wf2_ref, bf2_ref, lng_ref, lnb_ref, wfu1_ref, bfu1_ref,
               wfu2a_ref, wfu2b_ref, bfu2_ref, out_ref):
    kvm = kv_ref[...] * vm_ref[...][:, :, None]
    q_rows = q_ref[...]
    q = jnp.dot(q_rows, wq_ref[...], preferred_element_type=jnp.float32) + bq_ref[...]
    kv2 = kvm.reshape(M * BN, D_MODEL)
    k = jnp.dot(kv2, wk_ref[...], preferred_element_type=jnp.float32) + bk_ref[...]
    v = jnp.dot(kv2, wv_ref[...], preferred_element_type=jnp.float32) + bv_ref[...]
    k4 = k.reshape(M, BN, N_HEADS, HEAD_DIM)
    v4 = v.reshape(M, BN, N_HEADS, HEAD_DIM)
    q4 = q.reshape(1, BN, N_HEADS, HEAD_DIM)
    scores = (q4 * k4).sum(-1) * (1.0 / 4.0)
    smax = scores.max(axis=0, keepdims=True)
    e = jnp.exp(scores - smax)
    w = e / e.sum(axis=0, keepdims=True)
    attn = (w[:, :, :, None] * v4).sum(axis=0)
    o = jnp.dot(attn.reshape(BN, D_MODEL), wo_ref[...],
                preferred_element_type=jnp.float32) + bo_ref[...]
    h1 = jnp.maximum(
        jnp.dot(o, wf1_ref[...], preferred_element_type=jnp.float32)
        + bf1_ref[...], 0.0)
    t2 = jnp.dot(h1, wf2_ref[...], preferred_element_type=jnp.float32) + bf2_ref[...]
    t = o + t2
    mu = t.mean(axis=1, keepdims=True)
    var = ((t - mu) ** 2).mean(axis=1, keepdims=True)
    tgt = (t - mu) / jnp.sqrt(var + 1e-5) * lng_ref[...] + lnb_ref[...]
    tf = jnp.dot(tgt, wfu1_ref[...], preferred_element_type=jnp.float32) + bfu1_ref[...]
    out_ref[...] = (
        jnp.dot(feat_ref[...], wfu2a_ref[...], preferred_element_type=jnp.float32)
        + jnp.dot(tf, wfu2b_ref[...], preferred_element_type=jnp.float32)
        + bfu2_ref[...])


def _bn_body(x_ref, g_ref, b_ref, out_ref):
    x = x_ref[...]
    mu = x.mean(axis=0, keepdims=True)
    var = ((x - mu) ** 2).mean(axis=0, keepdims=True)
    y = (x - mu) / jnp.sqrt(var + 1e-5) * g_ref[...] + b_ref[...]
    out_ref[...] = jnp.maximum(y, 0.0)


def kernel(voxel_coords, voxel_features, W_proj, b_proj, W_pe1, b_pe1, W_pe2,
           b_pe2, Wq, bq, Wk, bk, Wv, bv, Wo, bo, W_f1, b_f1, W_f2, b_f2,
           ln_g, ln_b, W_fu1, b_fu1, W_fu2, b_fu2, bn_g, bn_b):
    crt = voxel_coords.astype(jnp.float32)
    ct = crt.T
    r2 = lambda a: a.reshape(1, -1)

    src = pl.pallas_call(
        _src_body,
        out_shape=jax.ShapeDtypeStruct((N, D_MODEL), jnp.float32),
    )(voxel_features, crt, W_proj, r2(b_proj), W_pe1, r2(b_pe1), W_pe2,
      r2(b_pe2))

    idx, validf = pl.pallas_call(
        _knn_body,
        grid=(N // BQ,),
        in_specs=[
            pl.BlockSpec((BQ, 3), lambda i: (i, 0)),
            pl.BlockSpec((3, N), lambda i: (0, 0)),
        ],
        out_specs=[
            pl.BlockSpec((BQ, M), lambda i: (i, 0)),
            pl.BlockSpec((BQ, M), lambda i: (i, 0)),
        ],
        out_shape=[
            jax.ShapeDtypeStruct((N, M), jnp.int32),
            jax.ShapeDtypeStruct((N, M), jnp.float32),
        ],
    )(crt, ct)

    neigh_flat = _gather_rows(src, idx.reshape(-1))
    kv_view = neigh_flat.reshape(M, N, D_MODEL)
    q_view = neigh_flat.reshape(N, M * D_MODEL)[:, :D_MODEL]
    vm_kv = validf.reshape(-1).reshape(M, N)

    fused_pre = pl.pallas_call(
        _attn_body,
        grid=(N // BN,),
        in_specs=[
            pl.BlockSpec((M, BN, D_MODEL), lambda i: (0, i, 0)),
            pl.BlockSpec((BN, D_MODEL), lambda i: (i, 0)),
            pl.BlockSpec((M, BN), lambda i: (0, i)),
            pl.BlockSpec((BN, D_CHL), lambda i: (i, 0)),
            pl.BlockSpec((D_MODEL, D_MODEL), lambda i: (0, 0)),
            pl.BlockSpec((1, D_MODEL), lambda i: (0, 0)),
            pl.BlockSpec((D_MODEL, D_MODEL), lambda i: (0, 0)),
            pl.BlockSpec((1, D_MODEL), lambda i: (0, 0)),
            pl.BlockSpec((D_MODEL, D_MODEL), lambda i: (0, 0)),
            pl.BlockSpec((1, D_MODEL), lambda i: (0, 0)),
            pl.BlockSpec((D_MODEL, D_MODEL), lambda i: (0, 0)),
            pl.BlockSpec((1, D_MODEL), lambda i: (0, 0)),
            pl.BlockSpec((D_MODEL, D_FFN), lambda i: (0, 0)),
            pl.BlockSpec((1, D_FFN), lambda i: (0, 0)),
            pl.BlockSpec((D_FFN, D_MODEL), lambda i: (0, 0)),
            pl.BlockSpec((1, D_MODEL), lambda i: (0, 0)),
            pl.BlockSpec((1, D_MODEL), lambda i: (0, 0)),
            pl.BlockSpec((1, D_MODEL), lambda i: (0, 0)),
            pl.BlockSpec((D_MODEL, D_CHL), lambda i: (0, 0)),
            pl.BlockSpec((1, D_CHL), lambda i: (0, 0)),
            pl.BlockSpec((D_CHL, D_CHL), lambda i: (0, 0)),
            pl.BlockSpec((D_CHL, D_CHL), lambda i: (0, 0)),
            pl.BlockSpec((1, D_CHL), lambda i: (0, 0)),
        ],
        out_specs=pl.BlockSpec((BN, D_CHL), lambda i: (i, 0)),
        out_shape=jax.ShapeDtypeStruct((N, D_CHL), jnp.float32),
    )(kv_view, q_view, vm_kv, voxel_features, Wq, r2(bq), Wk, r2(bk), Wv,
      r2(bv), Wo, r2(bo), W_f1, r2(b_f1), W_f2, r2(b_f2), r2(ln_g), r2(ln_b),
      W_fu1, r2(b_fu1), W_fu2[:D_CHL], W_fu2[D_CHL:], r2(b_fu2))

    return pl.pallas_call(
        _bn_body,
        out_shape=jax.ShapeDtypeStruct((N, D_CHL), jnp.float32),
    )(fused_pre, r2(bn_g), r2(bn_b))
